# Initial kernel scaffold; baseline (speedup 1.0000x reference)
import jax
import jax.numpy as jnp
from jax.experimental import pallas as pl


def kernel(Xi, Xv, fm_bias, W_first, W_second, W1, b1, W2, b2, W3, b3):
    raise NotImplementedError("write your pallas kernel here")


# speedup vs baseline: 16.7551x; 16.7551x over previous
import functools

import jax
import jax.numpy as jnp
from jax import lax
from jax.experimental import pallas as pl
from jax.experimental.pallas import tpu as pltpu
from jax.experimental.pallas import tpu_sc as plsc

B = 4096
F = 26
V = 1000
E = 128
D1 = 1024
D2 = 512

NUM_WORKERS = 32
SAMPLES_PER_WORKER = B // NUM_WORKERS
ROWS_PER_WORKER = SAMPLES_PER_WORKER * F
CHUNK = 128
NCHUNK = ROWS_PER_WORKER // CHUNK
SPAD = 32


def _sc_gather(table_hbm, idx2_hbm, idxs_hbm, wf_hbm,
               e2_out, s_out,
               idx_v, idxs_v, wf_v, s_v, rows0, rows1,
               gsem0, gsem1, ssem0, ssem1):
    wid = lax.axis_index("s") * 2 + lax.axis_index("c")
    base = wid * SAMPLES_PER_WORKER
    rbase = wid * ROWS_PER_WORKER

    pltpu.sync_copy(idx2_hbm.at[pl.ds(rbase, ROWS_PER_WORKER)], idx_v)
    pltpu.sync_copy(idxs_hbm.at[pl.ds(base * SPAD, SAMPLES_PER_WORKER * SPAD)],
                    idxs_v)
    pltpu.sync_copy(wf_hbm, wf_v)

    def s_body(i, carry):
        ids = idxs_v[pl.ds(i * 16, 16)]
        vals = plsc.load_gather(wf_v, [ids])
        s_v[pl.ds(i * 16, 16)] = vals
        return carry

    lax.fori_loop(0, SAMPLES_PER_WORKER * SPAD // 16, s_body, 0)
    pltpu.sync_copy(s_v, s_out.at[pl.ds(base * SPAD, SAMPLES_PER_WORKER * SPAD)])

    rows = (rows0, rows1)
    gsem = (gsem0, gsem1)
    ssem = (ssem0, ssem1)
    gcp = [None, None]
    scp = [None, None]

    def start_gather(c, p):
        idx_c = idx_v.at[pl.ds(c * CHUNK, CHUNK)]
        return pltpu.async_copy(table_hbm.at[idx_c], rows[p], gsem[p])

    gcp[0] = start_gather(0, 0)
    for c in range(NCHUNK):
        p = c % 2
        if c + 1 < NCHUNK:
            q = 1 - p
            if scp[q] is not None:
                scp[q].wait()
            gcp[q] = start_gather(c + 1, q)
        gcp[p].wait()
        scp[p] = pltpu.async_copy(
            rows[p], e2_out.at[pl.ds(rbase + c * CHUNK, CHUNK)], ssem[p])
    scp[0].wait()
    scp[1].wait()


def _mlp_body(fmb_ref, e2_ref, xv_ref, s_ref, w1s_ref, w1m_ref, w1r0_ref,
              b1_ref, w2_ref, b2_ref, w3_ref, b3_ref, out_ref):
    xv = xv_ref[:]
    e2 = e2_ref[:]
    scaled = jnp.concatenate(
        [e2[:, f * E:(f + 1) * E] * xv[:, f:f + 1] for f in range(F)], axis=1)
    acc = jnp.dot(scaled, w1s_ref[:], preferred_element_type=jnp.float32)
    acc += jnp.dot(s_ref[:] * xv, w1m_ref[:],
                   preferred_element_type=jnp.float32)
    h1 = jnp.tanh(acc + b1_ref[:] + fmb_ref[0, 0] * w1r0_ref[:])
    h2 = jnp.tanh(jnp.dot(h1, w2_ref[:], preferred_element_type=jnp.float32)
                  + b2_ref[:])
    out_ref[:] = (jnp.dot(h2, w3_ref[:], preferred_element_type=jnp.float32)
                  + b3_ref[0, 0])


def kernel(Xi, Xv, fm_bias, W_first, W_second, W1, b1, W2, b2, W3, b3):
    idx = Xi[:, :, 0].astype(jnp.int32)
    offs = (jnp.arange(F, dtype=jnp.int32) * V)[None, :]
    idx2 = (idx + offs).reshape(B * F)
    idxs = jnp.concatenate(
        [idx + offs,
         jnp.full((B, SPAD - F), F * V, dtype=jnp.int32)], axis=1
    ).reshape(B * SPAD)

    table = W_second.reshape(F * V, E)
    wf = jnp.concatenate(
        [W_first.reshape(F * V), jnp.zeros((8,), jnp.float32)])

    mesh = plsc.VectorSubcoreMesh(
        core_axis_name="c", subcore_axis_name="s", num_cores=2)
    sc = functools.partial(
        pl.kernel,
        mesh=mesh,
        compiler_params=pltpu.CompilerParams(needs_layout_passes=False),
        out_type=(
            jax.ShapeDtypeStruct((B * F, E), jnp.float32),
            jax.ShapeDtypeStruct((B * SPAD,), jnp.float32),
        ),
        scratch_types=[
            pltpu.VMEM((ROWS_PER_WORKER,), jnp.int32),
            pltpu.VMEM((SAMPLES_PER_WORKER * SPAD,), jnp.int32),
            pltpu.VMEM((F * V + 8,), jnp.float32),
            pltpu.VMEM((SAMPLES_PER_WORKER * SPAD,), jnp.float32),
            pltpu.VMEM((CHUNK, E), jnp.float32),
            pltpu.VMEM((CHUNK, E), jnp.float32),
            pltpu.SemaphoreType.DMA,
            pltpu.SemaphoreType.DMA,
            pltpu.SemaphoreType.DMA,
            pltpu.SemaphoreType.DMA,
        ],
    )(_sc_gather)
    e2_rows, s_flat = sc(table, idx2, idxs, wf)
    e2 = e2_rows.reshape(B, F * E)
    s = s_flat.reshape(B, SPAD)

    xvp = jnp.concatenate(
        [Xv, jnp.zeros((B, SPAD - F), jnp.float32)], axis=1)
    w1m = jnp.concatenate(
        [W1[1:1 + F], jnp.zeros((SPAD - F, D1), jnp.float32)], axis=0)
    w1s = W1[1 + F:]
    w1r0 = W1[0:1]

    BLK = 512
    grid = (B // BLK,)
    out = pl.pallas_call(
        _mlp_body,
        grid=grid,
        in_specs=[
            pl.BlockSpec(memory_space=pltpu.SMEM),
            pl.BlockSpec((BLK, F * E), lambda i: (i, 0)),
            pl.BlockSpec((BLK, SPAD), lambda i: (i, 0)),
            pl.BlockSpec((BLK, SPAD), lambda i: (i, 0)),
            pl.BlockSpec((F * E, D1), lambda i: (0, 0)),
            pl.BlockSpec((SPAD, D1), lambda i: (0, 0)),
            pl.BlockSpec((1, D1), lambda i: (0, 0)),
            pl.BlockSpec((1, D1), lambda i: (0, 0)),
            pl.BlockSpec((D1, D2), lambda i: (0, 0)),
            pl.BlockSpec((1, D2), lambda i: (0, 0)),
            pl.BlockSpec((D2, 1), lambda i: (0, 0)),
            pl.BlockSpec(memory_space=pltpu.SMEM),
        ],
        out_specs=pl.BlockSpec((BLK, 1), lambda i: (i, 0)),
        out_shape=jax.ShapeDtypeStruct((B, 1), jnp.float32),
    )(fm_bias.reshape(1, 1), e2, xvp, s, w1s, w1m, w1r0,
      b1.reshape(1, D1), W2, b2.reshape(1, D2), W3, b3.reshape(1, 1))
    return out


# speedup vs baseline: 22.8417x; 1.3633x over previous
import functools

import jax
import jax.numpy as jnp
from jax import lax
from jax.experimental import pallas as pl
from jax.experimental.pallas import tpu as pltpu
from jax.experimental.pallas import tpu_sc as plsc

B = 4096
F = 26
V = 1000
E = 128
D1 = 1024
D2 = 512

NUM_WORKERS = 32
SAMPLES_PER_WORKER = B // NUM_WORKERS
ROWS_PER_WORKER = SAMPLES_PER_WORKER * F
CHUNK = 128
NCHUNK = ROWS_PER_WORKER // CHUNK
SPAD = 32


def _sc_gather(table_hbm, idx2_hbm, idxs_hbm, wf_hbm,
               e2_out, s_out,
               idx_v, idxs_v, wf_v, s_v, rows0, rows1,
               gsem0, gsem1, ssem0, ssem1):
    wid = lax.axis_index("s") * 2 + lax.axis_index("c")
    base = wid * SAMPLES_PER_WORKER
    rbase = wid * ROWS_PER_WORKER

    pltpu.sync_copy(idx2_hbm.at[pl.ds(rbase, ROWS_PER_WORKER)], idx_v)
    pltpu.sync_copy(idxs_hbm.at[pl.ds(base * SPAD, SAMPLES_PER_WORKER * SPAD)],
                    idxs_v)
    pltpu.sync_copy(wf_hbm, wf_v)

    def s_body(i, carry):
        ids = idxs_v[pl.ds(i * 16, 16)]
        vals = plsc.load_gather(wf_v, [ids])
        s_v[pl.ds(i * 16, 16)] = vals
        return carry

    lax.fori_loop(0, SAMPLES_PER_WORKER * SPAD // 16, s_body, 0)
    pltpu.sync_copy(s_v, s_out.at[pl.ds(base * SPAD, SAMPLES_PER_WORKER * SPAD)])

    rows = (rows0, rows1)
    gsem = (gsem0, gsem1)
    ssem = (ssem0, ssem1)
    gcp = [None, None]
    scp = [None, None]

    def start_gather(c, p):
        idx_c = idx_v.at[pl.ds(c * CHUNK, CHUNK)]
        return pltpu.async_copy(table_hbm.at[idx_c], rows[p], gsem[p])

    gcp[0] = start_gather(0, 0)
    for c in range(NCHUNK):
        p = c % 2
        if c + 1 < NCHUNK:
            q = 1 - p
            if scp[q] is not None:
                scp[q].wait()
            gcp[q] = start_gather(c + 1, q)
        gcp[p].wait()
        scp[p] = pltpu.async_copy(
            rows[p],
            e2_out.at[pl.ds(base, SAMPLES_PER_WORKER), pl.ds(c * E, E)],
            ssem[p])
    scp[0].wait()
    scp[1].wait()


def _mlp_body(fmb_ref, e2_ref, xv_ref, s_ref, w1s_ref, w1m_ref, w1r0_ref,
              b1_ref, w2_ref, b2_ref, w3_ref, b3_ref, out_ref):
    xv = xv_ref[:]
    e2 = e2_ref[:]
    scaled = jnp.concatenate(
        [e2[:, f * E:(f + 1) * E] * xv[:, f:f + 1] for f in range(F)],
        axis=1).astype(jnp.bfloat16)
    acc = jnp.dot(scaled, w1s_ref[:], preferred_element_type=jnp.float32)
    acc += jnp.dot((s_ref[:] * xv).astype(jnp.bfloat16), w1m_ref[:],
                   preferred_element_type=jnp.float32)
    fmb = fmb_ref[0, 0].astype(jnp.bfloat16).astype(jnp.float32)
    w1r0 = w1r0_ref[:].astype(jnp.float32)
    h1 = jnp.tanh(acc + b1_ref[:] + fmb * w1r0)
    h2 = jnp.tanh(jnp.dot(h1.astype(jnp.bfloat16), w2_ref[:],
                          preferred_element_type=jnp.float32) + b2_ref[:])
    out_ref[:] = (jnp.dot(h2.astype(jnp.bfloat16), w3_ref[:],
                          preferred_element_type=jnp.float32) + b3_ref[0, 0])


def kernel(Xi, Xv, fm_bias, W_first, W_second, W1, b1, W2, b2, W3, b3):
    idx = Xi[:, :, 0].astype(jnp.int32)
    offs = (jnp.arange(F, dtype=jnp.int32) * V)[None, :]
    idx2 = (idx + offs).reshape(NUM_WORKERS, SAMPLES_PER_WORKER, F)
    idx2 = idx2.transpose(0, 2, 1).reshape(B * F)
    idxs = jnp.concatenate(
        [idx + offs,
         jnp.full((B, SPAD - F), F * V, dtype=jnp.int32)], axis=1
    ).reshape(B * SPAD)

    table = W_second.reshape(F * V, E)
    wf = jnp.concatenate(
        [W_first.reshape(F * V), jnp.zeros((8,), jnp.float32)])

    mesh = plsc.VectorSubcoreMesh(
        core_axis_name="c", subcore_axis_name="s", num_cores=2)
    sc = functools.partial(
        pl.kernel,
        mesh=mesh,
        compiler_params=pltpu.CompilerParams(needs_layout_passes=False),
        out_type=(
            jax.ShapeDtypeStruct((B, F * E), jnp.float32),
            jax.ShapeDtypeStruct((B * SPAD,), jnp.float32),
        ),
        scratch_types=[
            pltpu.VMEM((ROWS_PER_WORKER,), jnp.int32),
            pltpu.VMEM((SAMPLES_PER_WORKER * SPAD,), jnp.int32),
            pltpu.VMEM((F * V + 8,), jnp.float32),
            pltpu.VMEM((SAMPLES_PER_WORKER * SPAD,), jnp.float32),
            pltpu.VMEM((CHUNK, E), jnp.float32),
            pltpu.VMEM((CHUNK, E), jnp.float32),
            pltpu.SemaphoreType.DMA,
            pltpu.SemaphoreType.DMA,
            pltpu.SemaphoreType.DMA,
            pltpu.SemaphoreType.DMA,
        ],
    )(_sc_gather)
    e2, s_flat = sc(table, idx2, idxs, wf)
    s = s_flat.reshape(B, SPAD)

    xvp = jnp.concatenate(
        [Xv, jnp.zeros((B, SPAD - F), jnp.float32)], axis=1)
    w1m = jnp.concatenate(
        [W1[1:1 + F], jnp.zeros((SPAD - F, D1), jnp.float32)],
        axis=0).astype(jnp.bfloat16)
    w1s = W1[1 + F:].astype(jnp.bfloat16)
    w1r0 = W1[0:1].astype(jnp.bfloat16)
    w2b = W2.astype(jnp.bfloat16)
    w3b = W3.astype(jnp.bfloat16)

    BLK = 512
    grid = (B // BLK,)
    out = pl.pallas_call(
        _mlp_body,
        grid=grid,
        in_specs=[
            pl.BlockSpec(memory_space=pltpu.SMEM),
            pl.BlockSpec((BLK, F * E), lambda i: (i, 0)),
            pl.BlockSpec((BLK, SPAD), lambda i: (i, 0)),
            pl.BlockSpec((BLK, SPAD), lambda i: (i, 0)),
            pl.BlockSpec((F * E, D1), lambda i: (0, 0)),
            pl.BlockSpec((SPAD, D1), lambda i: (0, 0)),
            pl.BlockSpec((1, D1), lambda i: (0, 0)),
            pl.BlockSpec((1, D1), lambda i: (0, 0)),
            pl.BlockSpec((D1, D2), lambda i: (0, 0)),
            pl.BlockSpec((1, D2), lambda i: (0, 0)),
            pl.BlockSpec((D2, 1), lambda i: (0, 0)),
            pl.BlockSpec(memory_space=pltpu.SMEM),
        ],
        out_specs=pl.BlockSpec((BLK, 1), lambda i: (i, 0)),
        out_shape=jax.ShapeDtypeStruct((B, 1), jnp.float32),
    )(fm_bias.reshape(1, 1), e2, xvp, s, w1s, w1m, w1r0,
      b1.reshape(1, D1), w2b, b2.reshape(1, D2), w3b, b3.reshape(1, 1))
    return out


# speedup vs baseline: 23.3123x; 1.0206x over previous
import functools

import jax
import jax.numpy as jnp
from jax import lax
from jax.experimental import pallas as pl
from jax.experimental.pallas import tpu as pltpu
from jax.experimental.pallas import tpu_sc as plsc

B = 4096
F = 26
V = 1000
E = 128
D1 = 1024
D2 = 512

NUM_WORKERS = 32
SAMPLES_PER_WORKER = B // NUM_WORKERS
ROWS_PER_WORKER = SAMPLES_PER_WORKER * F
CHUNK = 128
NCHUNK = ROWS_PER_WORKER // CHUNK
SPAD = 32


NBUF = 4


def _sc_gather(table_hbm, idx2_hbm, idxs_hbm, wf_hbm, xv2_hbm,
               e2_out, s_out,
               idx_v, idxs_v, wf_v, s_v, xv_v,
               rows0, rows1, rows2, rows3,
               gsem0, gsem1, gsem2, gsem3,
               ssem0, ssem1, ssem2, ssem3):
    wid = lax.axis_index("s") * 2 + lax.axis_index("c")
    base = wid * SAMPLES_PER_WORKER
    rbase = wid * ROWS_PER_WORKER

    rows = (rows0, rows1, rows2, rows3)
    gsem = (gsem0, gsem1, gsem2, gsem3)
    ssem = (ssem0, ssem1, ssem2, ssem3)
    gcp = [None] * NBUF
    scp = [None] * NBUF

    pltpu.sync_copy(idx2_hbm.at[pl.ds(rbase, ROWS_PER_WORKER)], idx_v)

    def start_gather(c, p):
        idx_c = idx_v.at[pl.ds(c * CHUNK, CHUNK)]
        return pltpu.async_copy(table_hbm.at[idx_c], rows[p], gsem[p])

    gcp[0] = start_gather(0, 0)
    gcp[1] = start_gather(1, 1)

    pltpu.sync_copy(xv2_hbm.at[pl.ds(rbase, ROWS_PER_WORKER)], xv_v)
    pltpu.sync_copy(idxs_hbm.at[pl.ds(base * SPAD, SAMPLES_PER_WORKER * SPAD)],
                    idxs_v)
    pltpu.sync_copy(wf_hbm, wf_v)

    def s_body(i, carry):
        ids = idxs_v[pl.ds(i * 16, 16)]
        vals = plsc.load_gather(wf_v, [ids])
        s_v[pl.ds(i * 16, 16)] = vals
        return carry

    lax.fori_loop(0, SAMPLES_PER_WORKER * SPAD // 16, s_body, 0)
    pltpu.sync_copy(s_v, s_out.at[pl.ds(base * SPAD, SAMPLES_PER_WORKER * SPAD)])

    for c in range(NCHUNK):
        p = c % NBUF
        if c + 2 < NCHUNK:
            q = (c + 2) % NBUF
            if scp[q] is not None:
                scp[q].wait()
            gcp[q] = start_gather(c + 2, q)
        gcp[p].wait()

        buf = rows[p]

        def scale_row(r, carry, buf=buf, c=c):
            bc = plsc.load_gather(
                xv_v, [jnp.full((16,), c * CHUNK + r, jnp.int32)])
            for k in range(E // 16):
                buf[r, pl.ds(k * 16, 16)] = buf[r, pl.ds(k * 16, 16)] * bc
            return carry

        lax.fori_loop(0, CHUNK, scale_row, 0)
        scp[p] = pltpu.async_copy(
            buf,
            e2_out.at[pl.ds(base, SAMPLES_PER_WORKER), pl.ds(c * E, E)],
            ssem[p])
    for p in range(NBUF):
        if scp[p] is not None:
            scp[p].wait()


def _mlp_body(fmb_ref, e2_ref, xv_ref, s_ref, w1s_ref, w1m_ref, w1r0_ref,
              b1_ref, w2_ref, b2_ref, w3_ref, b3_ref, out_ref):
    xv = xv_ref[:]
    scaled = e2_ref[:].astype(jnp.bfloat16)
    acc = jnp.dot(scaled, w1s_ref[:], preferred_element_type=jnp.float32)
    acc += jnp.dot((s_ref[:] * xv).astype(jnp.bfloat16), w1m_ref[:],
                   preferred_element_type=jnp.float32)
    fmb = fmb_ref[0, 0].astype(jnp.bfloat16).astype(jnp.float32)
    w1r0 = w1r0_ref[:].astype(jnp.float32)
    h1 = jnp.tanh(acc + b1_ref[:] + fmb * w1r0)
    h2 = jnp.tanh(jnp.dot(h1.astype(jnp.bfloat16), w2_ref[:],
                          preferred_element_type=jnp.float32) + b2_ref[:])
    out_ref[:] = (jnp.dot(h2.astype(jnp.bfloat16), w3_ref[:],
                          preferred_element_type=jnp.float32) + b3_ref[0, 0])


def kernel(Xi, Xv, fm_bias, W_first, W_second, W1, b1, W2, b2, W3, b3):
    idx = Xi[:, :, 0].astype(jnp.int32)
    offs = (jnp.arange(F, dtype=jnp.int32) * V)[None, :]
    idx2 = (idx + offs).reshape(NUM_WORKERS, SAMPLES_PER_WORKER, F)
    idx2 = idx2.transpose(0, 2, 1).reshape(B * F)
    xv2 = Xv.reshape(NUM_WORKERS, SAMPLES_PER_WORKER, F)
    xv2 = xv2.transpose(0, 2, 1).reshape(B * F)
    idxs = jnp.concatenate(
        [idx + offs,
         jnp.full((B, SPAD - F), F * V, dtype=jnp.int32)], axis=1
    ).reshape(B * SPAD)

    table = W_second.reshape(F * V, E)
    wf = jnp.concatenate(
        [W_first.reshape(F * V), jnp.zeros((8,), jnp.float32)])

    mesh = plsc.VectorSubcoreMesh(
        core_axis_name="c", subcore_axis_name="s", num_cores=2)
    sc = functools.partial(
        pl.kernel,
        mesh=mesh,
        compiler_params=pltpu.CompilerParams(needs_layout_passes=False),
        out_type=(
            jax.ShapeDtypeStruct((B, F * E), jnp.float32),
            jax.ShapeDtypeStruct((B * SPAD,), jnp.float32),
        ),
        scratch_types=[
            pltpu.VMEM((ROWS_PER_WORKER,), jnp.int32),
            pltpu.VMEM((SAMPLES_PER_WORKER * SPAD,), jnp.int32),
            pltpu.VMEM((F * V + 8,), jnp.float32),
            pltpu.VMEM((SAMPLES_PER_WORKER * SPAD,), jnp.float32),
            pltpu.VMEM((ROWS_PER_WORKER,), jnp.float32),
            pltpu.VMEM((CHUNK, E), jnp.float32),
            pltpu.VMEM((CHUNK, E), jnp.float32),
            pltpu.VMEM((CHUNK, E), jnp.float32),
            pltpu.VMEM((CHUNK, E), jnp.float32),
            pltpu.SemaphoreType.DMA,
            pltpu.SemaphoreType.DMA,
            pltpu.SemaphoreType.DMA,
            pltpu.SemaphoreType.DMA,
            pltpu.SemaphoreType.DMA,
            pltpu.SemaphoreType.DMA,
            pltpu.SemaphoreType.DMA,
            pltpu.SemaphoreType.DMA,
        ],
    )(_sc_gather)
    e2, s_flat = sc(table, idx2, idxs, wf, xv2)
    s = s_flat.reshape(B, SPAD)

    xvp = jnp.concatenate(
        [Xv, jnp.zeros((B, SPAD - F), jnp.float32)], axis=1)
    w1m = jnp.concatenate(
        [W1[1:1 + F], jnp.zeros((SPAD - F, D1), jnp.float32)],
        axis=0).astype(jnp.bfloat16)
    w1s = W1[1 + F:].astype(jnp.bfloat16)
    w1r0 = W1[0:1].astype(jnp.bfloat16)
    w2b = W2.astype(jnp.bfloat16)
    w3b = W3.astype(jnp.bfloat16)

    BLK = 512
    grid = (B // BLK,)
    out = pl.pallas_call(
        _mlp_body,
        grid=grid,
        in_specs=[
            pl.BlockSpec(memory_space=pltpu.SMEM),
            pl.BlockSpec((BLK, F * E), lambda i: (i, 0)),
            pl.BlockSpec((BLK, SPAD), lambda i: (i, 0)),
            pl.BlockSpec((BLK, SPAD), lambda i: (i, 0)),
            pl.BlockSpec((F * E, D1), lambda i: (0, 0)),
            pl.BlockSpec((SPAD, D1), lambda i: (0, 0)),
            pl.BlockSpec((1, D1), lambda i: (0, 0)),
            pl.BlockSpec((1, D1), lambda i: (0, 0)),
            pl.BlockSpec((D1, D2), lambda i: (0, 0)),
            pl.BlockSpec((1, D2), lambda i: (0, 0)),
            pl.BlockSpec((D2, 1), lambda i: (0, 0)),
            pl.BlockSpec(memory_space=pltpu.SMEM),
        ],
        out_specs=pl.BlockSpec((BLK, 1), lambda i: (i, 0)),
        out_shape=jax.ShapeDtypeStruct((B, 1), jnp.float32),
    )(fm_bias.reshape(1, 1), e2, xvp, s, w1s, w1m, w1r0,
      b1.reshape(1, D1), w2b, b2.reshape(1, D2), w3b, b3.reshape(1, 1))
    return out
